# baseline (speedup 1.0000x reference)
import functools

import jax
import jax.numpy as jnp
from jax import lax
from jax.experimental import pallas as pl
from jax.experimental.pallas import tpu as pltpu
from jax.experimental.pallas import tpu_sc as plsc

NC = 2
NS = 16
NW = NC * NS
GS = 128
WB = 200
C = 32


def _mm_body(x_ref, w_ref, b_ref, z_ref, r_ref):
    xb = x_ref[...]
    big = jnp.dot(xb, w_ref[...], preferred_element_type=jnp.float32)
    bn8 = xb.shape[0]
    rowp = lax.broadcasted_iota(jnp.int32, (bn8, 2 * C), 0) % 8
    colq = (lax.broadcasted_iota(jnp.int32, (bn8, 2 * C), 1) % C) // 4
    big = jnp.where(rowp == colq, big, 0.0)
    zr = jnp.sum(big.reshape(bn8 // 8, 8, 2 * C), axis=1)
    z_ref[...] = zr[:, :C]
    r_ref[...] = zr[:, C:] + b_ref[...]


def _stage_a(x80, Wcat, bb):
    N8, K = x80.shape
    N = N8 // 8
    BN = 400
    grid = (N // BN,)
    return pl.pallas_call(
        _mm_body,
        grid=grid,
        in_specs=[
            pl.BlockSpec((BN * 8, K), lambda i: (i, 0)),
            pl.BlockSpec((K, 2 * C), lambda i: (0, 0)),
            pl.BlockSpec((1, C), lambda i: (0, 0)),
        ],
        out_specs=[
            pl.BlockSpec((BN, C), lambda i: (i, 0)),
            pl.BlockSpec((BN, C), lambda i: (i, 0)),
        ],
        out_shape=[
            jax.ShapeDtypeStruct((N, C), jnp.float32),
            jax.ShapeDtypeStruct((N, C), jnp.float32),
        ],
    )(x80, Wcat, bb)


def _make_sc_seg(N, NGRP):
    mesh = plsc.VectorSubcoreMesh(core_axis_name="c", subcore_axis_name="s")
    n_chunks = N // WB
    MAXG = pl.cdiv(NGRP, NW)
    NBUF = 6
    DEPTH = 3

    @functools.partial(
        pl.kernel,
        out_type=jax.ShapeDtypeStruct((NC, n_chunks, WB, C), jnp.float32),
        mesh=mesh,
        scratch_types=[
            pltpu.VMEM((MAXG, 2, GS), jnp.int32),
            pltpu.VMEM((MAXG, GS), jnp.float32),
            pltpu.VMEM((GS, C), jnp.float32),
            pltpu.VMEM((GS, C), jnp.float32),
            pltpu.VMEM((GS, C), jnp.float32),
            pltpu.VMEM((GS, C), jnp.float32),
            pltpu.VMEM((GS, C), jnp.float32),
            pltpu.VMEM((GS, C), jnp.float32),
            pltpu.VMEM((WB, C), jnp.float32),
            pltpu.VMEM_SHARED((N, C), jnp.float32),
            pltpu.SemaphoreType.DMA,
            pltpu.SemaphoreType.DMA,
        ],
        compiler_params=pltpu.CompilerParams(use_tc_tiling_on_sc=False),
    )
    def sc_seg(edges_hbm, ew_hbm, z_hbm, out_hbm,
               ed_v, ew_v, rows0, rows1, rows2, rows3, rows4, rows5, wb_v,
               acc_sh, gsem, ssem):
        c = lax.axis_index("c")
        s = lax.axis_index("s")
        wid = s * NC + c
        lo = (NGRP * wid) // NW
        ng = (NGRP * (wid + 1)) // NW - lo
        lanes = lax.iota(jnp.int32, 16)
        cnt1 = jnp.where(lanes == 3, 1.0, 0.0).astype(jnp.float32)
        zero16 = jnp.zeros((16,), jnp.float32)

        def zb(i, carry):
            wb_v[i, pl.ds(0, 16)] = zero16
            wb_v[i, pl.ds(16, 16)] = zero16
            return carry

        lax.fori_loop(0, WB, zb, 0)

        def zs(k, carry):
            ch = s + NS * k
            @pl.when(ch < n_chunks)
            def _():
                pltpu.sync_copy(wb_v, acc_sh.at[pl.ds(ch * WB, WB)])
            return carry

        lax.fori_loop(0, pl.cdiv(n_chunks, NS), zs, 0)

        pltpu.sync_copy(edges_hbm.at[pl.ds(lo, MAXG)], ed_v)
        pltpu.sync_copy(ew_hbm.at[pl.ds(lo, MAXG)], ew_v)

        plsc.subcore_barrier()

        bufs = (rows0, rows1, rows2, rows3, rows4, rows5)

        def scale(buf, g):
            def sub(bi, carry2):
                wvec = ew_v[g, pl.ds(bi * 16, 16)]
                for e in range(16):
                    wv = jnp.full((16,), wvec[e], jnp.float32)
                    r = bi * 16 + e
                    buf[r, pl.ds(0, 16)] = buf[r, pl.ds(0, 16)] * wv + cnt1
                    buf[r, pl.ds(16, 16)] = buf[r, pl.ds(16, 16)] * wv
                return carry2

            lax.fori_loop(0, GS // 16, sub, 0)

        def drain(sem, buf):
            pltpu.make_async_copy(z_hbm.at[pl.ds(0, GS)], buf, sem).wait()

        for j in range(DEPTH):
            pltpu.async_copy(z_hbm.at[ed_v.at[j, 0]], bufs[j], gsem)

        def ring(k, carry):
            for j in range(NBUF):
                g = NBUF * k + j
                cur = bufs[j]
                nxt = bufs[(j + DEPTH) % NBUF]

                @pl.when(g < ng)
                def _():
                    drain(gsem, cur)

                    @pl.when(g >= DEPTH)
                    def _():
                        drain(ssem, cur)

                    @pl.when(g + DEPTH < ng)
                    def _():
                        pltpu.async_copy(z_hbm.at[ed_v.at[g + DEPTH, 0]],
                                         nxt, gsem)

                    scale(cur, g)
                    pltpu.async_copy(cur, acc_sh.at[ed_v.at[g, 1]], ssem,
                                     add=True)
            return carry

        lax.fori_loop(0, pl.cdiv(MAXG, NBUF), ring, 0)
        for j in range(DEPTH):
            drain(ssem, bufs[j])

        plsc.subcore_barrier()

        def wb(k, carry):
            ch = s + NS * k
            @pl.when(ch < n_chunks)
            def _():
                pltpu.sync_copy(acc_sh.at[pl.ds(ch * WB, WB)], wb_v)
                pltpu.sync_copy(wb_v, out_hbm.at[c].at[ch])
            return carry

        lax.fori_loop(0, pl.cdiv(n_chunks, NS), wb, 0)

    return sc_seg


def _fin_body(p_ref, r_ref, g_ref, w_ref, bl_ref, o_ref):
    N = r_ref.shape[0]
    agg = p_ref[0] + p_ref[1]
    cnt = agg[:, 3:4]
    inv = 1.0 / jnp.maximum(cnt, 1.0)
    out = agg * inv + r_ref[...]
    j = lax.broadcasted_iota(jnp.int32, (N, C), 1)
    out = jnp.where((j & 3) != 3, out, 0.0)
    qi = lax.broadcasted_iota(jnp.int32, (C, C), 0) // 4
    qj = lax.broadcasted_iota(jnp.int32, (C, C), 1) // 4
    Gm = (qi == qj).astype(jnp.float32)
    ns = jnp.dot(out * out, Gm, preferred_element_type=jnp.float32)
    nrm = jnp.maximum(jnp.sqrt(ns), 1e-12)
    h = out / nrm
    h = h * (((j >> 2) + 1).astype(jnp.float32) * 0.125)
    h = jnp.where(h >= 0, h, 0.01 * h)
    logit = (
        jnp.dot(h, w_ref[...], preferred_element_type=jnp.float32)
        + bl_ref[0, 0]
        + g_ref[...]
    )
    m = jnp.max(logit)
    e = jnp.exp(logit - m)
    o_ref[...] = e / jnp.sum(e)


def _stage_c(partials, R, g, Wb, blin):
    N = R.shape[0]
    return pl.pallas_call(
        _fin_body,
        out_shape=jax.ShapeDtypeStruct((N, 1), jnp.float32),
    )(partials, R, g, Wb, blin)


def kernel(x, edge_index, edge_weight, Wl, Wr, b, W_lin, b_lin):
    N, FEAT, P = x.shape[1], x.shape[2], x.shape[3]
    E = edge_index.shape[1]
    MID = Wl.shape[2]

    x80 = x.transpose(0, 1, 3, 2).reshape(N * P, FEAT)
    Wl_pad = jnp.pad(Wl, ((0, 0), (0, 0), (0, 1)))
    Wr_pad = jnp.pad(Wr, ((0, 0), (0, 0), (0, 1)))
    Wcat = jnp.concatenate(
        [
            Wl_pad.transpose(1, 0, 2).reshape(FEAT, C),
            Wr_pad.transpose(1, 0, 2).reshape(FEAT, C),
        ],
        axis=1,
    )
    bb = jnp.pad(b, ((0, 0), (0, 1))).reshape(1, C)
    Wb = jnp.pad(W_lin.reshape(P, MID), ((0, 0), (0, 1))).reshape(C, 1)
    g = jax.random.gumbel(jax.random.key(42), (N,), jnp.float32)
    g = lax.optimization_barrier(g).reshape(N, 1)

    Z, R = _stage_a(x80, Wcat, bb)

    NGRP = E // GS
    edges = edge_index.reshape(2, NGRP, GS).transpose(1, 0, 2)
    ew2 = edge_weight.reshape(NGRP, GS)
    partials = _make_sc_seg(N, NGRP)(edges, ew2, Z)
    partials = partials.reshape(NC, N, C)

    return _stage_c(partials, R, g, Wb, b_lin.reshape(1, 1))


# speedup vs baseline: 1.0364x; 1.0364x over previous
import functools

import jax
import jax.numpy as jnp
from jax import lax
from jax.experimental import pallas as pl
from jax.experimental.pallas import tpu as pltpu
from jax.experimental.pallas import tpu_sc as plsc

NC = 2
NS = 16
NW = NC * NS
GS = 128
WB = 200
C = 32


def _mm_body(x_ref, w_ref, m_ref, b_ref, z_ref, r_ref):
    xb = x_ref[...]
    big = jnp.dot(xb, w_ref[...], preferred_element_type=jnp.float32)
    bn8 = xb.shape[0]
    big3 = big.reshape(bn8 // 8, 8, 2 * C) * m_ref[...][None]
    zr = jnp.sum(big3, axis=1)
    z_ref[...] = zr[:, :C]
    r_ref[...] = zr[:, C:] + b_ref[...]


def _stage_a(x80, Wcat, mask8, bb):
    N8, K = x80.shape
    N = N8 // 8
    BN = 1000
    grid = (N // BN,)
    return pl.pallas_call(
        _mm_body,
        grid=grid,
        in_specs=[
            pl.BlockSpec((BN * 8, K), lambda i: (i, 0)),
            pl.BlockSpec((K, 2 * C), lambda i: (0, 0)),
            pl.BlockSpec((8, 2 * C), lambda i: (0, 0)),
            pl.BlockSpec((1, C), lambda i: (0, 0)),
        ],
        out_specs=[
            pl.BlockSpec((BN, C), lambda i: (i, 0)),
            pl.BlockSpec((BN, C), lambda i: (i, 0)),
        ],
        out_shape=[
            jax.ShapeDtypeStruct((N, C), jnp.float32),
            jax.ShapeDtypeStruct((N, C), jnp.float32),
        ],
    )(x80, Wcat, mask8, bb)


def _make_sc_seg(N, NGRP):
    mesh = plsc.VectorSubcoreMesh(core_axis_name="c", subcore_axis_name="s")
    n_chunks = N // WB
    MAXG = pl.cdiv(NGRP, NW)
    NBUF = 4
    DEPTH = 2

    @functools.partial(
        pl.kernel,
        out_type=jax.ShapeDtypeStruct((NC, n_chunks, WB, C), jnp.float32),
        mesh=mesh,
        scratch_types=[
            pltpu.VMEM((MAXG, 2, GS), jnp.int32),
            pltpu.VMEM((MAXG, GS), jnp.float32),
            pltpu.VMEM((GS, C), jnp.float32),
            pltpu.VMEM((GS, C), jnp.float32),
            pltpu.VMEM((GS, C), jnp.float32),
            pltpu.VMEM((GS, C), jnp.float32),
            pltpu.VMEM((WB, C), jnp.float32),
            pltpu.VMEM_SHARED((N, C), jnp.float32),
            pltpu.SemaphoreType.DMA,
            pltpu.SemaphoreType.DMA,
        ],
        compiler_params=pltpu.CompilerParams(use_tc_tiling_on_sc=False),
    )
    def sc_seg(edges_hbm, ew_hbm, z_hbm, out_hbm,
               ed_v, ew_v, rows0, rows1, rows2, rows3, wb_v,
               acc_sh, gsem, ssem):
        c = lax.axis_index("c")
        s = lax.axis_index("s")
        wid = s * NC + c
        lo = (NGRP * wid) // NW
        ng = (NGRP * (wid + 1)) // NW - lo
        lanes = lax.iota(jnp.int32, 16)
        cnt1 = jnp.where(lanes == 3, 1.0, 0.0).astype(jnp.float32)
        zero16 = jnp.zeros((16,), jnp.float32)

        def zb(i, carry):
            wb_v[i, pl.ds(0, 16)] = zero16
            wb_v[i, pl.ds(16, 16)] = zero16
            return carry

        lax.fori_loop(0, WB, zb, 0)

        def zs(k, carry):
            ch = s + NS * k
            @pl.when(ch < n_chunks)
            def _():
                pltpu.sync_copy(wb_v, acc_sh.at[pl.ds(ch * WB, WB)])
            return carry

        lax.fori_loop(0, pl.cdiv(n_chunks, NS), zs, 0)

        pltpu.sync_copy(edges_hbm.at[pl.ds(lo, MAXG)], ed_v)
        pltpu.sync_copy(ew_hbm.at[pl.ds(lo, MAXG)], ew_v)

        plsc.subcore_barrier()

        bufs = (rows0, rows1, rows2, rows3)

        def scale(buf, g):
            def sub(bi, carry2):
                wvec = ew_v[g, pl.ds(bi * 16, 16)]
                for e in range(16):
                    wv = jnp.full((16,), wvec[e], jnp.float32)
                    r = bi * 16 + e
                    buf[r, pl.ds(0, 16)] = buf[r, pl.ds(0, 16)] * wv + cnt1
                    buf[r, pl.ds(16, 16)] = buf[r, pl.ds(16, 16)] * wv
                return carry2

            lax.fori_loop(0, GS // 16, sub, 0)

        def drain(sem, buf):
            pltpu.make_async_copy(z_hbm.at[pl.ds(0, GS)], buf, sem).wait()

        for j in range(DEPTH):
            pltpu.async_copy(z_hbm.at[ed_v.at[j, 0]], bufs[j], gsem)

        def ring(k, carry):
            for j in range(NBUF):
                g = NBUF * k + j
                cur = bufs[j]
                nxt = bufs[(j + DEPTH) % NBUF]

                @pl.when(g < ng)
                def _():
                    drain(gsem, cur)

                    @pl.when(g >= DEPTH)
                    def _():
                        drain(ssem, cur)

                    @pl.when(g + DEPTH < ng)
                    def _():
                        pltpu.async_copy(z_hbm.at[ed_v.at[g + DEPTH, 0]],
                                         nxt, gsem)

                    scale(cur, g)
                    pltpu.async_copy(cur, acc_sh.at[ed_v.at[g, 1]], ssem,
                                     add=True)
            return carry

        lax.fori_loop(0, pl.cdiv(MAXG, NBUF), ring, 0)
        for j in range(DEPTH):
            drain(ssem, bufs[j])

        plsc.subcore_barrier()

        def wb(k, carry):
            ch = s + NS * k
            @pl.when(ch < n_chunks)
            def _():
                pltpu.sync_copy(acc_sh.at[pl.ds(ch * WB, WB)], wb_v)
                pltpu.sync_copy(wb_v, out_hbm.at[c].at[ch])
            return carry

        lax.fori_loop(0, pl.cdiv(n_chunks, NS), wb, 0)

    return sc_seg


def _fin_body(p_ref, r_ref, g_ref, gm_ref, mask_ref, sw_ref, bl_ref, o_ref):
    agg = p_ref[0] + p_ref[1]
    cnt = agg[:, 3:4]
    inv = 1.0 / jnp.maximum(cnt, 1.0)
    out = (agg * inv + r_ref[...]) * mask_ref[...]
    ns = jnp.dot(out * out, gm_ref[...], preferred_element_type=jnp.float32)
    h = out * lax.rsqrt(jnp.maximum(ns, 1e-24))
    h = jnp.where(h >= 0, h, 0.01 * h)
    score = jnp.sum(h * sw_ref[...], axis=1)
    logit = score + bl_ref[0, 0] + g_ref[...]
    m = jnp.max(logit)
    e = jnp.exp(logit - m)
    o_ref[...] = e / jnp.sum(e)


def _stage_c(partials, R, g, Gm, mask_row, sw_row, blin):
    N = R.shape[0]
    return pl.pallas_call(
        _fin_body,
        out_shape=jax.ShapeDtypeStruct((N,), jnp.float32),
    )(partials, R, g, Gm, mask_row, sw_row, blin)


def kernel(x, edge_index, edge_weight, Wl, Wr, b, W_lin, b_lin):
    N, FEAT, P = x.shape[1], x.shape[2], x.shape[3]
    E = edge_index.shape[1]
    MID = Wl.shape[2]

    x80 = x.transpose(0, 1, 3, 2).reshape(N * P, FEAT)
    Wl_pad = jnp.pad(Wl, ((0, 0), (0, 0), (0, 1)))
    Wr_pad = jnp.pad(Wr, ((0, 0), (0, 0), (0, 1)))
    Wcat = jnp.concatenate(
        [
            Wl_pad.transpose(1, 0, 2).reshape(FEAT, C),
            Wr_pad.transpose(1, 0, 2).reshape(FEAT, C),
        ],
        axis=1,
    )
    bb = jnp.pad(b, ((0, 0), (0, 1))).reshape(1, C)
    qs = jnp.arange(C) // 4
    ms = jnp.arange(C) % 4
    mask8 = (jnp.arange(P)[:, None] == jnp.tile(qs, 2)[None, :]).astype(
        jnp.float32
    )
    Gm = (qs[:, None] == qs[None, :]).astype(jnp.float32)
    mask_row = (ms != 3).astype(jnp.float32).reshape(1, C)
    W_row = jnp.pad(W_lin.reshape(P, MID), ((0, 0), (0, 1))).reshape(1, C)
    sw_row = W_row * ((qs + 1).astype(jnp.float32) / P).reshape(1, C)
    g = jax.random.gumbel(jax.random.key(42), (N,), jnp.float32)
    g = lax.optimization_barrier(g)

    Z, R = _stage_a(x80, Wcat, mask8, bb)

    NGRP = E // GS
    edges = edge_index.reshape(2, NGRP, GS).transpose(1, 0, 2)
    ew2 = edge_weight.reshape(NGRP, GS)
    partials = _make_sc_seg(N, NGRP)(edges, ew2, Z)
    partials = partials.reshape(NC, N, C)

    out = _stage_c(partials, R, g, Gm, mask_row, sw_row, b_lin.reshape(1, 1))
    return out.reshape(N, 1)


# speedup vs baseline: 1.0723x; 1.0346x over previous
import functools

import jax
import jax.numpy as jnp
from jax import lax
from jax.experimental import pallas as pl
from jax.experimental.pallas import tpu as pltpu
from jax.experimental.pallas import tpu_sc as plsc

NC = 2
NS = 16
NW = NC * NS
GS = 128
WB = 200
C = 32


def _mm_body(x_ref, w_ref, m_ref, b_ref, z_ref, r_ref):
    xb = x_ref[...]
    big = jnp.dot(xb, w_ref[...], preferred_element_type=jnp.float32)
    bn8 = xb.shape[0]
    big3 = big.reshape(bn8 // 8, 8, 2 * C) * m_ref[...][None]
    zr = jnp.sum(big3, axis=1)
    z_ref[...] = zr[:, :C]
    r_ref[...] = zr[:, C:] + b_ref[...]


def _stage_a(x80, Wcat, mask8, bb):
    N8, K = x80.shape
    N = N8 // 8
    BN = 1000
    grid = (N // BN,)
    return pl.pallas_call(
        _mm_body,
        grid=grid,
        in_specs=[
            pl.BlockSpec((BN * 8, K), lambda i: (i, 0)),
            pl.BlockSpec((K, 2 * C), lambda i: (0, 0)),
            pl.BlockSpec((8, 2 * C), lambda i: (0, 0)),
            pl.BlockSpec((1, C), lambda i: (0, 0)),
        ],
        out_specs=[
            pl.BlockSpec((BN, C), lambda i: (i, 0)),
            pl.BlockSpec((BN, C), lambda i: (i, 0)),
        ],
        out_shape=[
            jax.ShapeDtypeStruct((N, C), jnp.float32),
            jax.ShapeDtypeStruct((N, C), jnp.float32),
        ],
    )(x80, Wcat, mask8, bb)


def _make_sc_seg(N, NGRP):
    mesh = plsc.VectorSubcoreMesh(core_axis_name="c", subcore_axis_name="s")
    n_chunks = N // WB
    MAXG = pl.cdiv(NGRP, NW)
    NBUF = 4
    DEPTH = 2

    @functools.partial(
        pl.kernel,
        out_type=jax.ShapeDtypeStruct((NC, n_chunks, WB, C), jnp.float32),
        mesh=mesh,
        scratch_types=[
            pltpu.VMEM((MAXG, 2, GS), jnp.int32),
            pltpu.VMEM((MAXG, GS), jnp.float32),
            pltpu.VMEM((GS, C), jnp.float32),
            pltpu.VMEM((GS, C), jnp.float32),
            pltpu.VMEM((GS, C), jnp.float32),
            pltpu.VMEM((GS, C), jnp.float32),
            pltpu.VMEM((WB, C), jnp.float32),
            pltpu.VMEM_SHARED((N, C), jnp.float32),
            pltpu.SemaphoreType.DMA,
            pltpu.SemaphoreType.DMA,
            pltpu.SemaphoreType.DMA,
            pltpu.SemaphoreType.DMA,
        ],
        compiler_params=pltpu.CompilerParams(use_tc_tiling_on_sc=False),
    )
    def sc_seg(edges_hbm, ew_hbm, z_hbm, out_hbm,
               ed_v, ew_v, rows0, rows1, rows2, rows3, wb_v,
               acc_sh, gsem0, gsem1, ssem0, ssem1):
        c = lax.axis_index("c")
        s = lax.axis_index("s")
        wid = s * NC + c
        lo = (NGRP * wid) // NW
        ng = (NGRP * (wid + 1)) // NW - lo
        lanes = lax.iota(jnp.int32, 16)
        cnt1 = jnp.where(lanes == 3, 1.0, 0.0).astype(jnp.float32)
        zero16 = jnp.zeros((16,), jnp.float32)

        def zb(i, carry):
            wb_v[i, pl.ds(0, 16)] = zero16
            wb_v[i, pl.ds(16, 16)] = zero16
            return carry

        lax.fori_loop(0, WB, zb, 0)

        def zs(k, carry):
            ch = s + NS * k
            @pl.when(ch < n_chunks)
            def _():
                pltpu.sync_copy(wb_v, acc_sh.at[pl.ds(ch * WB, WB)])
            return carry

        lax.fori_loop(0, pl.cdiv(n_chunks, NS), zs, 0)

        pltpu.sync_copy(edges_hbm.at[pl.ds(lo, MAXG)], ed_v)
        pltpu.sync_copy(ew_hbm.at[pl.ds(lo, MAXG)], ew_v)

        plsc.subcore_barrier()

        bufs = (rows0, rows1, rows2, rows3)

        def scale(buf, g):
            def sub(bi, carry2):
                wvec = ew_v[g, pl.ds(bi * 16, 16)]
                for e in range(16):
                    wv = jnp.full((16,), wvec[e], jnp.float32)
                    r = bi * 16 + e
                    buf[r, pl.ds(0, 16)] = buf[r, pl.ds(0, 16)] * wv + cnt1
                    buf[r, pl.ds(16, 16)] = buf[r, pl.ds(16, 16)] * wv
                return carry2

            lax.fori_loop(0, GS // 16, sub, 0)

        gsems = (gsem0, gsem1)
        ssems = (ssem0, ssem1)

        def drain(sem, buf):
            pltpu.make_async_copy(z_hbm.at[pl.ds(0, GS)], buf, sem).wait()

        for j in range(DEPTH):
            pltpu.async_copy(z_hbm.at[ed_v.at[j, 0]], bufs[j], gsems[j % 2])

        def ring(k, carry):
            for j in range(NBUF):
                g = NBUF * k + j
                cur = bufs[j]
                nxt = bufs[(j + DEPTH) % NBUF]

                @pl.when(g < ng)
                def _():
                    @pl.when(g >= DEPTH)
                    def _():
                        drain(ssems[j % 2], cur)

                    @pl.when(g + DEPTH < ng)
                    def _():
                        pltpu.async_copy(z_hbm.at[ed_v.at[g + DEPTH, 0]],
                                         nxt, gsems[j % 2])

                    drain(gsems[j % 2], cur)
                    scale(cur, g)
                    pltpu.async_copy(cur, acc_sh.at[ed_v.at[g, 1]],
                                     ssems[j % 2], add=True)
            return carry

        lax.fori_loop(0, pl.cdiv(MAXG, NBUF), ring, 0)
        drain(ssems[0], rows0)
        drain(ssems[1], rows1)

        plsc.subcore_barrier()

        def wb(k, carry):
            ch = s + NS * k
            @pl.when(ch < n_chunks)
            def _():
                pltpu.sync_copy(acc_sh.at[pl.ds(ch * WB, WB)], wb_v)
                pltpu.sync_copy(wb_v, out_hbm.at[c].at[ch])
            return carry

        lax.fori_loop(0, pl.cdiv(n_chunks, NS), wb, 0)

    return sc_seg


def _fin_body(p_ref, r_ref, g_ref, gm_ref, mask_ref, sw_ref, bl_ref, o_ref):
    agg = p_ref[0] + p_ref[1]
    cnt = agg[:, 3:4]
    inv = 1.0 / jnp.maximum(cnt, 1.0)
    out = (agg * inv + r_ref[...]) * mask_ref[...]
    ns = jnp.dot(out * out, gm_ref[...], preferred_element_type=jnp.float32)
    h = out * lax.rsqrt(jnp.maximum(ns, 1e-24))
    h = jnp.where(h >= 0, h, 0.01 * h)
    score = jnp.sum(h * sw_ref[...], axis=1)
    logit = score + bl_ref[0, 0] + g_ref[...]
    m = jnp.max(logit)
    e = jnp.exp(logit - m)
    o_ref[...] = e / jnp.sum(e)


def _stage_c(partials, R, g, Gm, mask_row, sw_row, blin):
    N = R.shape[0]
    return pl.pallas_call(
        _fin_body,
        out_shape=jax.ShapeDtypeStruct((N,), jnp.float32),
    )(partials, R, g, Gm, mask_row, sw_row, blin)


def kernel(x, edge_index, edge_weight, Wl, Wr, b, W_lin, b_lin):
    N, FEAT, P = x.shape[1], x.shape[2], x.shape[3]
    E = edge_index.shape[1]
    MID = Wl.shape[2]

    x80 = x.transpose(0, 1, 3, 2).reshape(N * P, FEAT)
    Wl_pad = jnp.pad(Wl, ((0, 0), (0, 0), (0, 1)))
    Wr_pad = jnp.pad(Wr, ((0, 0), (0, 0), (0, 1)))
    Wcat = jnp.concatenate(
        [
            Wl_pad.transpose(1, 0, 2).reshape(FEAT, C),
            Wr_pad.transpose(1, 0, 2).reshape(FEAT, C),
        ],
        axis=1,
    )
    bb = jnp.pad(b, ((0, 0), (0, 1))).reshape(1, C)
    qs = jnp.arange(C) // 4
    ms = jnp.arange(C) % 4
    mask8 = (jnp.arange(P)[:, None] == jnp.tile(qs, 2)[None, :]).astype(
        jnp.float32
    )
    Gm = (qs[:, None] == qs[None, :]).astype(jnp.float32)
    mask_row = (ms != 3).astype(jnp.float32).reshape(1, C)
    W_row = jnp.pad(W_lin.reshape(P, MID), ((0, 0), (0, 1))).reshape(1, C)
    sw_row = W_row * ((qs + 1).astype(jnp.float32) / P).reshape(1, C)
    g = jax.random.gumbel(jax.random.key(42), (N,), jnp.float32)
    g = lax.optimization_barrier(g)

    Z, R = _stage_a(x80, Wcat, mask8, bb)

    NGRP = E // GS
    edges = edge_index.reshape(2, NGRP, GS).transpose(1, 0, 2)
    ew2 = edge_weight.reshape(NGRP, GS)
    partials = _make_sc_seg(N, NGRP)(edges, ew2, Z)
    partials = partials.reshape(NC, N, C)

    out = _stage_c(partials, R, g, Gm, mask_row, sw_row, b_lin.reshape(1, 1))
    return out.reshape(N, 1)


# speedup vs baseline: 1.0839x; 1.0109x over previous
import functools

import jax
import jax.numpy as jnp
from jax import lax
from jax.experimental import pallas as pl
from jax.experimental.pallas import tpu as pltpu
from jax.experimental.pallas import tpu_sc as plsc

NC = 2
NS = 16
NW = NC * NS
GS = 128
WB = 200
C = 32


def _mm_body(x_ref, w_ref, m_ref, b_ref, z_ref, r_ref):
    xb = x_ref[...]
    big = jnp.dot(xb, w_ref[...], preferred_element_type=jnp.float32)
    bn8 = xb.shape[0]
    big3 = big.reshape(bn8 // 8, 8, 2 * C) * m_ref[...][None]
    zr = jnp.sum(big3, axis=1)
    z_ref[...] = zr[:, :C]
    r_ref[...] = zr[:, C:] + b_ref[...]


def _stage_a(x80, Wcat, mask8, bb):
    N8, K = x80.shape
    N = N8 // 8
    BN = 1000
    grid = (N // BN,)
    return pl.pallas_call(
        _mm_body,
        grid=grid,
        in_specs=[
            pl.BlockSpec((BN * 8, K), lambda i: (i, 0)),
            pl.BlockSpec((K, 2 * C), lambda i: (0, 0)),
            pl.BlockSpec((8, 2 * C), lambda i: (0, 0)),
            pl.BlockSpec((1, C), lambda i: (0, 0)),
        ],
        out_specs=[
            pl.BlockSpec((BN, C), lambda i: (i, 0)),
            pl.BlockSpec((BN, C), lambda i: (i, 0)),
        ],
        out_shape=[
            jax.ShapeDtypeStruct((N, C), jnp.float32),
            jax.ShapeDtypeStruct((N, C), jnp.float32),
        ],
    )(x80, Wcat, mask8, bb)


def _make_sc_seg(N, NGRP):
    mesh = plsc.VectorSubcoreMesh(core_axis_name="c", subcore_axis_name="s")
    n_chunks = N // WB
    MAXG = pl.cdiv(NGRP, NW)
    NBUF = 8
    DEPTH = 4

    @functools.partial(
        pl.kernel,
        out_type=jax.ShapeDtypeStruct((NC, n_chunks, WB, C), jnp.float32),
        mesh=mesh,
        scratch_types=[
            pltpu.VMEM((MAXG, 2, GS), jnp.int32),
            pltpu.VMEM((MAXG, GS), jnp.float32),
            pltpu.VMEM((GS, C), jnp.float32),
            pltpu.VMEM((GS, C), jnp.float32),
            pltpu.VMEM((GS, C), jnp.float32),
            pltpu.VMEM((GS, C), jnp.float32),
            pltpu.VMEM((GS, C), jnp.float32),
            pltpu.VMEM((GS, C), jnp.float32),
            pltpu.VMEM((GS, C), jnp.float32),
            pltpu.VMEM((GS, C), jnp.float32),
            pltpu.VMEM((WB, C), jnp.float32),
            pltpu.VMEM_SHARED((N, C), jnp.float32),
            pltpu.SemaphoreType.DMA,
            pltpu.SemaphoreType.DMA,
            pltpu.SemaphoreType.DMA,
            pltpu.SemaphoreType.DMA,
            pltpu.SemaphoreType.DMA,
            pltpu.SemaphoreType.DMA,
            pltpu.SemaphoreType.DMA,
            pltpu.SemaphoreType.DMA,
        ],
        compiler_params=pltpu.CompilerParams(use_tc_tiling_on_sc=False),
    )
    def sc_seg(edges_hbm, ew_hbm, z_hbm, out_hbm,
               ed_v, ew_v, rows0, rows1, rows2, rows3, rows4, rows5, rows6,
               rows7, wb_v, acc_sh, gsem0, gsem1, gsem2, gsem3,
               ssem0, ssem1, ssem2, ssem3):
        c = lax.axis_index("c")
        s = lax.axis_index("s")
        wid = s * NC + c
        lo = (NGRP * wid) // NW
        ng = (NGRP * (wid + 1)) // NW - lo
        lanes = lax.iota(jnp.int32, 16)
        cnt1 = jnp.where(lanes == 3, 1.0, 0.0).astype(jnp.float32)
        zero16 = jnp.zeros((16,), jnp.float32)

        def zb(i, carry):
            wb_v[i, pl.ds(0, 16)] = zero16
            wb_v[i, pl.ds(16, 16)] = zero16
            return carry

        lax.fori_loop(0, WB, zb, 0)

        def zs(k, carry):
            ch = s + NS * k
            @pl.when(ch < n_chunks)
            def _():
                pltpu.sync_copy(wb_v, acc_sh.at[pl.ds(ch * WB, WB)])
            return carry

        lax.fori_loop(0, pl.cdiv(n_chunks, NS), zs, 0)

        pltpu.sync_copy(edges_hbm.at[pl.ds(lo, MAXG)], ed_v)
        pltpu.sync_copy(ew_hbm.at[pl.ds(lo, MAXG)], ew_v)

        plsc.subcore_barrier()

        bufs = (rows0, rows1, rows2, rows3, rows4, rows5, rows6, rows7)

        def scale(buf, g):
            def sub(bi, carry2):
                wvec = ew_v[g, pl.ds(bi * 16, 16)]
                for e in range(16):
                    wv = jnp.full((16,), wvec[e], jnp.float32)
                    r = bi * 16 + e
                    buf[r, pl.ds(0, 16)] = buf[r, pl.ds(0, 16)] * wv + cnt1
                    buf[r, pl.ds(16, 16)] = buf[r, pl.ds(16, 16)] * wv
                return carry2

            lax.fori_loop(0, GS // 16, sub, 0)

        gsems = (gsem0, gsem1, gsem2, gsem3)
        ssems = (ssem0, ssem1, ssem2, ssem3)

        def drain(sem, buf):
            pltpu.make_async_copy(z_hbm.at[pl.ds(0, GS)], buf, sem).wait()

        for j in range(DEPTH):
            pltpu.async_copy(z_hbm.at[ed_v.at[j, 0]], bufs[j], gsems[j % 4])

        def ring(k, carry):
            for j in range(NBUF):
                g = NBUF * k + j
                cur = bufs[j]
                nxt = bufs[(j + DEPTH) % NBUF]

                @pl.when(g < ng)
                def _():
                    @pl.when(g >= DEPTH)
                    def _():
                        drain(ssems[j % 4], cur)

                    @pl.when(g + DEPTH < ng)
                    def _():
                        pltpu.async_copy(z_hbm.at[ed_v.at[g + DEPTH, 0]],
                                         nxt, gsems[j % 4])

                    drain(gsems[j % 4], cur)
                    scale(cur, g)
                    pltpu.async_copy(cur, acc_sh.at[ed_v.at[g, 1]],
                                     ssems[j % 4], add=True)
            return carry

        lax.fori_loop(0, pl.cdiv(MAXG, NBUF), ring, 0)
        for j in range(DEPTH):
            drain(ssems[j], bufs[j])

        plsc.subcore_barrier()

        def wb(k, carry):
            ch = s + NS * k
            @pl.when(ch < n_chunks)
            def _():
                pltpu.sync_copy(acc_sh.at[pl.ds(ch * WB, WB)], wb_v)
                pltpu.sync_copy(wb_v, out_hbm.at[c].at[ch])
            return carry

        lax.fori_loop(0, pl.cdiv(n_chunks, NS), wb, 0)

    return sc_seg


def _fin_body(p_ref, r_ref, g_ref, gm_ref, mask_ref, sw_ref, bl_ref, o_ref):
    agg = p_ref[0] + p_ref[1]
    cnt = agg[:, 3:4]
    inv = 1.0 / jnp.maximum(cnt, 1.0)
    out = (agg * inv + r_ref[...]) * mask_ref[...]
    ns = jnp.dot(out * out, gm_ref[...], preferred_element_type=jnp.float32)
    h = out * lax.rsqrt(jnp.maximum(ns, 1e-24))
    h = jnp.where(h >= 0, h, 0.01 * h)
    score = jnp.sum(h * sw_ref[...], axis=1)
    logit = score + bl_ref[0, 0] + g_ref[...]
    m = jnp.max(logit)
    e = jnp.exp(logit - m)
    o_ref[...] = e / jnp.sum(e)


def _stage_c(partials, R, g, Gm, mask_row, sw_row, blin):
    N = R.shape[0]
    return pl.pallas_call(
        _fin_body,
        out_shape=jax.ShapeDtypeStruct((N,), jnp.float32),
    )(partials, R, g, Gm, mask_row, sw_row, blin)


def kernel(x, edge_index, edge_weight, Wl, Wr, b, W_lin, b_lin):
    N, FEAT, P = x.shape[1], x.shape[2], x.shape[3]
    E = edge_index.shape[1]
    MID = Wl.shape[2]

    x80 = x.transpose(0, 1, 3, 2).reshape(N * P, FEAT)
    Wl_pad = jnp.pad(Wl, ((0, 0), (0, 0), (0, 1)))
    Wr_pad = jnp.pad(Wr, ((0, 0), (0, 0), (0, 1)))
    Wcat = jnp.concatenate(
        [
            Wl_pad.transpose(1, 0, 2).reshape(FEAT, C),
            Wr_pad.transpose(1, 0, 2).reshape(FEAT, C),
        ],
        axis=1,
    )
    bb = jnp.pad(b, ((0, 0), (0, 1))).reshape(1, C)
    qs = jnp.arange(C) // 4
    ms = jnp.arange(C) % 4
    mask8 = (jnp.arange(P)[:, None] == jnp.tile(qs, 2)[None, :]).astype(
        jnp.float32
    )
    Gm = (qs[:, None] == qs[None, :]).astype(jnp.float32)
    mask_row = (ms != 3).astype(jnp.float32).reshape(1, C)
    W_row = jnp.pad(W_lin.reshape(P, MID), ((0, 0), (0, 1))).reshape(1, C)
    sw_row = W_row * ((qs + 1).astype(jnp.float32) / P).reshape(1, C)
    g = jax.random.gumbel(jax.random.key(42), (N,), jnp.float32)
    g = lax.optimization_barrier(g)

    Z, R = _stage_a(x80, Wcat, mask8, bb)

    NGRP = E // GS
    edges = edge_index.reshape(2, NGRP, GS).transpose(1, 0, 2)
    ew2 = edge_weight.reshape(NGRP, GS)
    partials = _make_sc_seg(N, NGRP)(edges, ew2, Z)
    partials = partials.reshape(NC, N, C)

    out = _stage_c(partials, R, g, Gm, mask_row, sw_row, b_lin.reshape(1, 1))
    return out.reshape(N, 1)
